# baseline (speedup 1.0000x reference)
import functools

import jax
import jax.numpy as jnp
from jax import lax
from jax.experimental import pallas as pl
from jax.experimental.pallas import tpu as pltpu
from jax.experimental.pallas import tpu_sc as plsc

N = 10000
E = 320000
DIM = 128
EMB = 64
FIXED = 34
NTYPES = 44

NC = 2
NS = 16
L = 16
NW = NC * NS

EB = 64
K = 4
NB = 160
EPW = NB * EB
EPAD = NW * EPW
NPAD = 10240
RPW = NPAD // NS
CPW = 632
NHP = NS * CPW

RB = 1000
NRB = N // RB


@functools.cache
def _build_sc_degree():
    mesh = plsc.VectorSubcoreMesh(core_axis_name="c", subcore_axis_name="s")
    return functools.partial(
        pl.kernel, mesh=mesh,
        out_type=jax.ShapeDtypeStruct((NC, NHP, L), jnp.float32),
        scratch_types=[
            pltpu.VMEM((NB, 2, EB), jnp.int32),
            pltpu.VMEM((EB, L), jnp.float32),
            pltpu.VMEM_SHARED((NPAD, L), jnp.float32),
            pltpu.SemaphoreType.DMA,
        ],
    )(_sc_degree_body)


def _sc_degree(edges):
    return _build_sc_degree()(edges)


def _sc_degree_body(edges_hbm, out_hbm, idx_v, buf_v, acc_sh, sem):
    c = lax.axis_index("c")
    s = lax.axis_index("s")
    w = c * NS + s

    @pl.loop(0, EB)
    def _(r):
        buf_v[r, :] = jnp.zeros((L,), jnp.float32)

    @pl.loop(0, RPW // EB)
    def _(j):
        pltpu.sync_copy(buf_v, acc_sh.at[pl.ds(s * RPW + j * EB, EB)])

    pltpu.sync_copy(edges_hbm.at[w], idx_v)
    plsc.subcore_barrier()

    @pl.loop(0, EB)
    def _(r):
        buf_v[r, :] = jnp.ones((L,), jnp.float32)

    @pl.loop(0, NB)
    def _(b):
        pltpu.make_async_copy(buf_v, acc_sh.at[idx_v.at[b, 1]], sem).start()

    @pl.loop(0, NB)
    def _(b):
        pltpu.make_async_copy(buf_v, acc_sh.at[idx_v.at[b, 1]], sem).wait()

    plsc.subcore_barrier()
    row = pl.multiple_of(s * CPW, 8)
    pltpu.sync_copy(acc_sh.at[pl.ds(row, CPW)],
                    out_hbm.at[c, pl.ds(row, CPW)])


@functools.cache
def _build_sc_aggregate():
    mesh = plsc.VectorSubcoreMesh(core_axis_name="c", subcore_axis_name="s")
    return functools.partial(
        pl.kernel, mesh=mesh,
        out_type=jax.ShapeDtypeStruct((NC, NHP, DIM), jnp.float32),
        scratch_types=(
            [pltpu.VMEM((2, EB), jnp.int32) for _ in range(K)]
            + [pltpu.VMEM((EB, DIM), jnp.float32) for _ in range(K)]
            + [pltpu.SemaphoreType.DMA for _ in range(2 * K)]
            + [pltpu.VMEM_SHARED((NPAD, DIM), jnp.float32)]
        ),
    )(_sc_aggregate_body)


def _sc_aggregate(hws, edges):
    return _build_sc_aggregate()(hws, edges)


def _sc_aggregate_body(hws_hbm, edges_hbm, out_hbm, *scratch):
    idx = scratch[:K]
    rows = scratch[K:2 * K]
    gsem = scratch[2 * K:3 * K]
    isem = scratch[3 * K:4 * K]
    acc_sh = scratch[4 * K]
    r0 = rows[0]
    c = lax.axis_index("c")
    s = lax.axis_index("s")
    w = c * NS + s

    def idx_start(b, j):
        pltpu.make_async_copy(edges_hbm.at[w, b], idx[j], isem[j]).start()

    def idx_wait(b, j):
        pltpu.make_async_copy(edges_hbm.at[w, b], idx[j], isem[j]).wait()

    def gather_start(j):
        pltpu.make_async_copy(hws_hbm.at[idx[j].at[0]], rows[j], gsem[j]).start()

    def gather_wait(j):
        pltpu.make_async_copy(hws_hbm.at[idx[j].at[0]], rows[j], gsem[j]).wait()

    def scatter(j):
        pltpu.sync_copy(rows[j], acc_sh.at[idx[j].at[1]], add=True)

    @pl.loop(0, EB)
    def _(r):
        @pl.loop(0, DIM // L)
        def _(j):
            r0[r, pl.ds(j * L, L)] = jnp.zeros((L,), jnp.float32)

    @pl.loop(0, RPW // EB)
    def _(j):
        pltpu.sync_copy(r0, acc_sh.at[pl.ds(s * RPW + j * EB, EB)])

    for j in range(K - 1):
        pltpu.sync_copy(edges_hbm.at[w, j], idx[j])
    plsc.subcore_barrier()

    for j in range(K - 1):
        gather_start(j)
    idx_start(K - 1, K - 1)

    @pl.loop(0, NB - K, step=K)
    def _(g):
        for j in range(K):
            b = g + j
            jp = (j - 1) % K
            gather_wait(j)
            idx_wait(b + K - 1, jp)
            gather_start(jp)
            scatter(j)
            idx_start(b + K, j)

    jlast = (NB - 1) % K
    idx_wait(NB - 1, jlast)
    gather_start(jlast)
    for b in range(NB - K, NB):
        j = b % K
        gather_wait(j)
        scatter(j)

    plsc.subcore_barrier()
    row = pl.multiple_of(s * CPW, 8)
    pltpu.sync_copy(acc_sh.at[pl.ds(row, CPW)],
                    out_hbm.at[c, pl.ds(row, CPW)])


def _tc_front_body(x_ref, emb_ref, wl_ref, bl_ref, hist_ref, w1_ref,
                   hws_ref, dinv_ref):
    xb = x_ref[...]
    xt = xb[:, :NTYPES]
    m = jnp.max(xt, axis=1, keepdims=True)
    iota = lax.broadcasted_iota(jnp.int32, xt.shape, 1)
    idx = jnp.min(jnp.where(xt == m, iota, NTYPES), axis=1, keepdims=True)
    onehot = (iota == idx).astype(jnp.float32)
    table = jnp.dot(emb_ref[...], wl_ref[:EMB, :],
                    preferred_element_type=jnp.float32)
    h = jnp.dot(onehot, table, preferred_element_type=jnp.float32)
    h = h + jnp.dot(xb[:, NTYPES:], wl_ref[EMB:, :],
                    preferred_element_type=jnp.float32)
    h = jnp.maximum(h + bl_ref[...], 0.0)
    hw = jnp.dot(h, w1_ref[...], preferred_element_type=jnp.float32)
    deg = hist_ref[0, :, :1] + hist_ref[1, :, :1] + 1.0
    dinv = lax.rsqrt(deg)
    hws_ref[...] = hw * dinv
    dinv_ref[...] = dinv


def _tc_front(x, atom_emb, W_lin, b_lin, hist, W1):
    return pl.pallas_call(
        _tc_front_body,
        grid=(NRB,),
        in_specs=[
            pl.BlockSpec((RB, NTYPES + FIXED), lambda i: (i, 0)),
            pl.BlockSpec((NTYPES, EMB), lambda i: (0, 0)),
            pl.BlockSpec((EMB + FIXED, DIM), lambda i: (0, 0)),
            pl.BlockSpec((1, DIM), lambda i: (0, 0)),
            pl.BlockSpec((NC, RB, L), lambda i: (0, i, 0)),
            pl.BlockSpec((DIM, DIM), lambda i: (0, 0)),
        ],
        out_specs=[
            pl.BlockSpec((RB, DIM), lambda i: (i, 0)),
            pl.BlockSpec((RB, 1), lambda i: (i, 0)),
        ],
        out_shape=[
            jax.ShapeDtypeStruct((N, DIM), jnp.float32),
            jax.ShapeDtypeStruct((N, 1), jnp.float32),
        ],
    )(x, atom_emb, W_lin, b_lin.reshape(1, DIM), hist, W1)


def _tc_mid_body(a_ref, hws_ref, dinv_ref, b_ref, w_ref, out_ref):
    dinv = dinv_ref[...]
    h = dinv * (a_ref[0] + a_ref[1] + hws_ref[...]) + b_ref[...]
    h = jnp.maximum(h, 0.0)
    out_ref[...] = jnp.dot(h, w_ref[...], preferred_element_type=jnp.float32) * dinv


def _tc_mid(acc, hws, dinv, b, W):
    return pl.pallas_call(
        _tc_mid_body,
        grid=(NRB,),
        in_specs=[
            pl.BlockSpec((NC, RB, DIM), lambda i: (0, i, 0)),
            pl.BlockSpec((RB, DIM), lambda i: (i, 0)),
            pl.BlockSpec((RB, 1), lambda i: (i, 0)),
            pl.BlockSpec((1, DIM), lambda i: (0, 0)),
            pl.BlockSpec((DIM, DIM), lambda i: (0, 0)),
        ],
        out_specs=pl.BlockSpec((RB, DIM), lambda i: (i, 0)),
        out_shape=jax.ShapeDtypeStruct((N, DIM), jnp.float32),
    )(acc, hws, dinv, b.reshape(1, DIM), W)


def _tc_final_body(a_ref, hws_ref, dinv_ref, b_ref, out_ref):
    h = dinv_ref[...] * (a_ref[0] + a_ref[1] + hws_ref[...]) + b_ref[...]
    out_ref[...] = jnp.maximum(h, 0.0)


def _tc_final(acc, hws, dinv, b):
    return pl.pallas_call(
        _tc_final_body,
        grid=(NRB,),
        in_specs=[
            pl.BlockSpec((NC, RB, DIM), lambda i: (0, i, 0)),
            pl.BlockSpec((RB, DIM), lambda i: (i, 0)),
            pl.BlockSpec((RB, 1), lambda i: (i, 0)),
            pl.BlockSpec((1, DIM), lambda i: (0, 0)),
        ],
        out_specs=pl.BlockSpec((RB, DIM), lambda i: (i, 0)),
        out_shape=jax.ShapeDtypeStruct((N, DIM), jnp.float32),
    )(acc, hws, dinv, b.reshape(1, DIM))


def kernel(x, edge_index, batch, atom_emb, W_lin, b_lin, W1, b1, W2, b2):
    del batch
    pad = EPAD - E
    pad_src = (jnp.arange(pad, dtype=jnp.int32) * 127) % N
    pad_dst = N + (jnp.arange(pad, dtype=jnp.int32) % (NPAD - N))
    src_r = jnp.concatenate([edge_index[0], pad_src])
    dst_r = jnp.concatenate([edge_index[1], pad_dst])
    edges = jnp.stack([src_r.reshape(NW, NB, EB), dst_r.reshape(NW, NB, EB)],
                      axis=2)

    hist = _sc_degree(edges)
    hws1, dinv = _tc_front(x, atom_emb, W_lin, b_lin, hist, W1)
    acc1 = _sc_aggregate(hws1, edges)
    hws2 = _tc_mid(acc1, hws1, dinv, b1, W2)
    acc2 = _sc_aggregate(hws2, edges)
    return _tc_final(acc2, hws2, dinv, b2)


# speedup vs baseline: 1.1993x; 1.1993x over previous
import functools

import jax
import jax.numpy as jnp
from jax import lax
from jax.experimental import pallas as pl
from jax.experimental.pallas import tpu as pltpu
from jax.experimental.pallas import tpu_sc as plsc

N = 10000
E = 320000
DIM = 128
EMB = 64
FIXED = 34
NTYPES = 44

NC = 2
NS = 16
L = 16
NW = NC * NS

EB = 128
K = 2
NB = 80
EPW = NB * EB
EPAD = NW * EPW
NPAD = 10240
RPW = NPAD // NS
CPW = 632
NHP = NS * CPW

RB = 1000
NRB = N // RB


@functools.cache
def _build_sc_degree():
    mesh = plsc.VectorSubcoreMesh(core_axis_name="c", subcore_axis_name="s")
    return functools.partial(
        pl.kernel, mesh=mesh,
        out_type=jax.ShapeDtypeStruct((NC, NHP, L), jnp.float32),
        scratch_types=[
            pltpu.VMEM((NB, 2, EB), jnp.int32),
            pltpu.VMEM((EB, L), jnp.float32),
            pltpu.VMEM_SHARED((NPAD, L), jnp.float32),
            pltpu.SemaphoreType.DMA,
        ],
    )(_sc_degree_body)


def _sc_degree(edges):
    return _build_sc_degree()(edges)


def _sc_degree_body(edges_hbm, out_hbm, idx_v, buf_v, acc_sh, sem):
    c = lax.axis_index("c")
    s = lax.axis_index("s")
    w = c * NS + s

    @pl.loop(0, EB)
    def _(r):
        buf_v[r, :] = jnp.zeros((L,), jnp.float32)

    @pl.loop(0, RPW // EB)
    def _(j):
        pltpu.sync_copy(buf_v, acc_sh.at[pl.ds(s * RPW + j * EB, EB)])

    pltpu.sync_copy(edges_hbm.at[w], idx_v)
    plsc.subcore_barrier()

    @pl.loop(0, EB)
    def _(r):
        buf_v[r, :] = jnp.ones((L,), jnp.float32)

    @pl.loop(0, NB)
    def _(b):
        pltpu.make_async_copy(buf_v, acc_sh.at[idx_v.at[b, 1]], sem).start()

    @pl.loop(0, NB)
    def _(b):
        pltpu.make_async_copy(buf_v, acc_sh.at[idx_v.at[b, 1]], sem).wait()

    plsc.subcore_barrier()
    row = pl.multiple_of(s * CPW, 8)
    pltpu.sync_copy(acc_sh.at[pl.ds(row, CPW)],
                    out_hbm.at[c, pl.ds(row, CPW)])


@functools.cache
def _build_sc_aggregate():
    mesh = plsc.VectorSubcoreMesh(core_axis_name="c", subcore_axis_name="s")
    return functools.partial(
        pl.kernel, mesh=mesh,
        out_type=jax.ShapeDtypeStruct((NC, NHP, DIM), jnp.float32),
        scratch_types=(
            [pltpu.VMEM((2, EB), jnp.int32) for _ in range(K)]
            + [pltpu.VMEM((EB, DIM), jnp.float32) for _ in range(K)]
            + [pltpu.SemaphoreType.DMA for _ in range(2 * K)]
            + [pltpu.VMEM_SHARED((NPAD, DIM), jnp.float32)]
        ),
    )(_sc_aggregate_body)


def _sc_aggregate(hws, edges):
    return _build_sc_aggregate()(hws, edges)


def _sc_aggregate_body(hws_hbm, edges_hbm, out_hbm, *scratch):
    idx = scratch[:K]
    rows = scratch[K:2 * K]
    gsem = scratch[2 * K:3 * K]
    isem = scratch[3 * K:4 * K]
    acc_sh = scratch[4 * K]
    c = lax.axis_index("c")
    s = lax.axis_index("s")
    w = c * NS + s

    def idx_start(b, j):
        pltpu.make_async_copy(edges_hbm.at[w, b], idx[j], isem[j]).start()

    def idx_wait(b, j):
        pltpu.make_async_copy(edges_hbm.at[w, b], idx[j], isem[j]).wait()

    def gather_start(j):
        pltpu.make_async_copy(hws_hbm.at[idx[j].at[0]], rows[j], gsem[j]).start()

    def gather_wait(j):
        pltpu.make_async_copy(hws_hbm.at[idx[j].at[0]], rows[j], gsem[j]).wait()

    def scatter(j):
        pltpu.sync_copy(rows[j], acc_sh.at[idx[j].at[1]], add=True)

    for j in range(K - 1):
        pltpu.sync_copy(edges_hbm.at[w, j], idx[j])
    for j in range(K - 1):
        gather_start(j)

    zb = rows[K - 1]

    @pl.loop(0, EB)
    def _(r):
        @pl.loop(0, DIM // L)
        def _(j):
            zb[r, pl.ds(j * L, L)] = jnp.zeros((L,), jnp.float32)

    @pl.loop(0, RPW // EB)
    def _(j):
        pltpu.sync_copy(zb, acc_sh.at[pl.ds(s * RPW + j * EB, EB)])

    plsc.subcore_barrier()
    idx_start(K - 1, K - 1)

    @pl.loop(0, NB - K, step=K)
    def _(g):
        for j in range(K):
            b = g + j
            jp = (j - 1) % K
            gather_wait(j)
            idx_wait(b + K - 1, jp)
            gather_start(jp)
            scatter(j)
            idx_start(b + K, j)

    jlast = (NB - 1) % K
    idx_wait(NB - 1, jlast)
    gather_start(jlast)
    for b in range(NB - K, NB):
        j = b % K
        gather_wait(j)
        scatter(j)

    plsc.subcore_barrier()
    row = pl.multiple_of(s * CPW, 8)
    pltpu.sync_copy(acc_sh.at[pl.ds(row, CPW)],
                    out_hbm.at[c, pl.ds(row, CPW)])


def _tc_front_a_body(x_ref, emb_ref, wl_ref, bl_ref, w1_ref, hw_ref):
    xb = x_ref[...]
    xt = xb[:, :NTYPES]
    m = jnp.max(xt, axis=1, keepdims=True)
    iota = lax.broadcasted_iota(jnp.int32, xt.shape, 1)
    idx = jnp.min(jnp.where(xt == m, iota, NTYPES), axis=1, keepdims=True)
    onehot = (iota == idx).astype(jnp.float32)
    table = jnp.dot(emb_ref[...], wl_ref[:EMB, :],
                    preferred_element_type=jnp.float32)
    h = jnp.dot(onehot, table, preferred_element_type=jnp.float32)
    h = h + jnp.dot(xb[:, NTYPES:], wl_ref[EMB:, :],
                    preferred_element_type=jnp.float32)
    h = jnp.maximum(h + bl_ref[...], 0.0)
    hw_ref[...] = jnp.dot(h, w1_ref[...], preferred_element_type=jnp.float32)


def _tc_front_a(x, atom_emb, W_lin, b_lin, W1):
    return pl.pallas_call(
        _tc_front_a_body,
        grid=(NRB,),
        in_specs=[
            pl.BlockSpec((RB, NTYPES + FIXED), lambda i: (i, 0)),
            pl.BlockSpec((NTYPES, EMB), lambda i: (0, 0)),
            pl.BlockSpec((EMB + FIXED, DIM), lambda i: (0, 0)),
            pl.BlockSpec((1, DIM), lambda i: (0, 0)),
            pl.BlockSpec((DIM, DIM), lambda i: (0, 0)),
        ],
        out_specs=pl.BlockSpec((RB, DIM), lambda i: (i, 0)),
        out_shape=jax.ShapeDtypeStruct((N, DIM), jnp.float32),
    )(x, atom_emb, W_lin, b_lin.reshape(1, DIM), W1)


def _tc_front_b_body(hist_ref, hw_ref, hws_ref, dinv_ref):
    deg = hist_ref[0, :, :1] + hist_ref[1, :, :1] + 1.0
    dinv = lax.rsqrt(deg)
    hws_ref[...] = hw_ref[...] * dinv
    dinv_ref[...] = dinv


def _tc_front_b(hist, hw):
    return pl.pallas_call(
        _tc_front_b_body,
        grid=(NRB,),
        in_specs=[
            pl.BlockSpec((NC, RB, L), lambda i: (0, i, 0)),
            pl.BlockSpec((RB, DIM), lambda i: (i, 0)),
        ],
        out_specs=[
            pl.BlockSpec((RB, DIM), lambda i: (i, 0)),
            pl.BlockSpec((RB, 1), lambda i: (i, 0)),
        ],
        out_shape=[
            jax.ShapeDtypeStruct((N, DIM), jnp.float32),
            jax.ShapeDtypeStruct((N, 1), jnp.float32),
        ],
    )(hist, hw)


def _tc_mid_body(a_ref, hws_ref, dinv_ref, b_ref, w_ref, out_ref):
    dinv = dinv_ref[...]
    h = dinv * (a_ref[0] + a_ref[1] + hws_ref[...]) + b_ref[...]
    h = jnp.maximum(h, 0.0)
    out_ref[...] = jnp.dot(h, w_ref[...], preferred_element_type=jnp.float32) * dinv


def _tc_mid(acc, hws, dinv, b, W):
    return pl.pallas_call(
        _tc_mid_body,
        grid=(NRB,),
        in_specs=[
            pl.BlockSpec((NC, RB, DIM), lambda i: (0, i, 0)),
            pl.BlockSpec((RB, DIM), lambda i: (i, 0)),
            pl.BlockSpec((RB, 1), lambda i: (i, 0)),
            pl.BlockSpec((1, DIM), lambda i: (0, 0)),
            pl.BlockSpec((DIM, DIM), lambda i: (0, 0)),
        ],
        out_specs=pl.BlockSpec((RB, DIM), lambda i: (i, 0)),
        out_shape=jax.ShapeDtypeStruct((N, DIM), jnp.float32),
    )(acc, hws, dinv, b.reshape(1, DIM), W)


def _tc_final_body(a_ref, hws_ref, dinv_ref, b_ref, out_ref):
    h = dinv_ref[...] * (a_ref[0] + a_ref[1] + hws_ref[...]) + b_ref[...]
    out_ref[...] = jnp.maximum(h, 0.0)


def _tc_final(acc, hws, dinv, b):
    return pl.pallas_call(
        _tc_final_body,
        grid=(NRB,),
        in_specs=[
            pl.BlockSpec((NC, RB, DIM), lambda i: (0, i, 0)),
            pl.BlockSpec((RB, DIM), lambda i: (i, 0)),
            pl.BlockSpec((RB, 1), lambda i: (i, 0)),
            pl.BlockSpec((1, DIM), lambda i: (0, 0)),
        ],
        out_specs=pl.BlockSpec((RB, DIM), lambda i: (i, 0)),
        out_shape=jax.ShapeDtypeStruct((N, DIM), jnp.float32),
    )(acc, hws, dinv, b.reshape(1, DIM))


def kernel(x, edge_index, batch, atom_emb, W_lin, b_lin, W1, b1, W2, b2):
    del batch
    pad = EPAD - E
    pad_src = (jnp.arange(pad, dtype=jnp.int32) * 127) % N
    pad_dst = N + (jnp.arange(pad, dtype=jnp.int32) % (NPAD - N))
    src_r = jnp.concatenate([edge_index[0], pad_src])
    dst_r = jnp.concatenate([edge_index[1], pad_dst])
    edges = jnp.stack([src_r.reshape(NW, NB, EB), dst_r.reshape(NW, NB, EB)],
                      axis=2)

    hw1 = _tc_front_a(x, atom_emb, W_lin, b_lin, W1)
    hist = _sc_degree(edges)
    hws1, dinv = _tc_front_b(hist, hw1)
    acc1 = _sc_aggregate(hws1, edges)
    hws2 = _tc_mid(acc1, hws1, dinv, b1, W2)
    acc2 = _sc_aggregate(hws2, edges)
    return _tc_final(acc2, hws2, dinv, b2)


# speedup vs baseline: 1.2155x; 1.0135x over previous
import functools

import jax
import jax.numpy as jnp
from jax import lax
from jax.experimental import pallas as pl
from jax.experimental.pallas import tpu as pltpu
from jax.experimental.pallas import tpu_sc as plsc

N = 10000
E = 320000
DIM = 128
EMB = 64
FIXED = 34
NTYPES = 44

NC = 2
NS = 16
L = 16
NW = NC * NS

EB = 128
K = 2
NB = 80
EPW = NB * EB
EPAD = NW * EPW
NPAD = 10240
RPW = NPAD // NS
CPW = 632
NHP = NS * CPW

RB = 1000
NRB = N // RB


@functools.cache
def _build_sc_degree():
    mesh = plsc.VectorSubcoreMesh(core_axis_name="c", subcore_axis_name="s")
    return functools.partial(
        pl.kernel, mesh=mesh,
        out_type=jax.ShapeDtypeStruct((NC, NHP, L), jnp.float32),
        scratch_types=[
            pltpu.VMEM((NB, 2, EB), jnp.int32),
            pltpu.VMEM((EB, L), jnp.float32),
            pltpu.VMEM_SHARED((NPAD, L), jnp.float32),
            pltpu.SemaphoreType.DMA,
        ],
    )(_sc_degree_body)


def _sc_degree(edges):
    return _build_sc_degree()(edges)


def _sc_degree_body(edges_hbm, out_hbm, idx_v, buf_v, acc_sh, sem):
    c = lax.axis_index("c")
    s = lax.axis_index("s")
    w = c * NS + s

    @pl.loop(0, EB)
    def _(r):
        buf_v[r, :] = jnp.zeros((L,), jnp.float32)

    @pl.loop(0, RPW // EB)
    def _(j):
        pltpu.sync_copy(buf_v, acc_sh.at[pl.ds(s * RPW + j * EB, EB)])

    pltpu.sync_copy(edges_hbm.at[w], idx_v)
    plsc.subcore_barrier()

    @pl.loop(0, EB)
    def _(r):
        buf_v[r, :] = jnp.ones((L,), jnp.float32)

    @pl.loop(0, NB)
    def _(b):
        pltpu.make_async_copy(buf_v, acc_sh.at[idx_v.at[b, 1]], sem).start()

    @pl.loop(0, NB)
    def _(b):
        pltpu.make_async_copy(buf_v, acc_sh.at[idx_v.at[b, 1]], sem).wait()

    plsc.subcore_barrier()
    row = pl.multiple_of(s * CPW, 8)
    pltpu.sync_copy(acc_sh.at[pl.ds(row, CPW)],
                    out_hbm.at[c, pl.ds(row, CPW)])


@functools.cache
def _build_sc_aggregate():
    mesh = plsc.VectorSubcoreMesh(core_axis_name="c", subcore_axis_name="s")
    return functools.partial(
        pl.kernel, mesh=mesh,
        out_type=jax.ShapeDtypeStruct((NC, NHP, DIM), jnp.float32),
        scratch_types=(
            [pltpu.VMEM((2, EB), jnp.int32) for _ in range(4)]
            + [pltpu.VMEM((EB, DIM), jnp.float32) for _ in range(2)]
            + [pltpu.SemaphoreType.DMA for _ in range(8)]
            + [pltpu.VMEM_SHARED((NPAD, DIM), jnp.float32)]
        ),
    )(_sc_aggregate_body)


def _sc_aggregate(hws, edges):
    return _build_sc_aggregate()(hws, edges)


def _sc_aggregate_body(hws_hbm, edges_hbm, out_hbm, *scratch):
    idx = scratch[:4]
    rows = scratch[4:6]
    isem = scratch[6:10]
    gsem = scratch[10:12]
    ssem = scratch[12:14]
    acc_sh = scratch[14]
    c = lax.axis_index("c")
    s = lax.axis_index("s")
    w = c * NS + s

    def idx_start(b, i):
        pltpu.make_async_copy(edges_hbm.at[w, b], idx[i], isem[i]).start()

    def idx_wait(b, i):
        pltpu.make_async_copy(edges_hbm.at[w, b], idx[i], isem[i]).wait()

    def gather_start(i, j):
        pltpu.make_async_copy(hws_hbm.at[idx[i].at[0]], rows[j], gsem[j]).start()

    def gather_wait(i, j):
        pltpu.make_async_copy(hws_hbm.at[idx[i].at[0]], rows[j], gsem[j]).wait()

    def scat_start(i, j):
        pltpu.make_async_copy(rows[j], acc_sh.at[idx[i].at[1]], ssem[j]).start()

    def scat_wait(i, j):
        pltpu.make_async_copy(rows[j], acc_sh.at[idx[i].at[1]], ssem[j]).wait()

    def body(b, i4, j2, first=False, has_g=True, has_i=True, last=False):
        jn = 1 - j2
        gather_wait(i4, j2)
        if not first:
            scat_wait((i4 + 3) % 4, jn)
        if has_g:
            idx_wait(b + 1, (i4 + 1) % 4)
            gather_start((i4 + 1) % 4, jn)
        scat_start(i4, j2)
        if has_i:
            idx_start(b + 2, (i4 + 2) % 4)
        if last:
            scat_wait(i4, j2)

    idx_start(0, 0)
    idx_start(1, 1)
    idx_wait(0, 0)
    gather_start(0, 0)

    zb = rows[1]

    @pl.loop(0, EB)
    def _(r):
        @pl.loop(0, DIM // L)
        def _(j):
            zb[r, pl.ds(j * L, L)] = jnp.zeros((L,), jnp.float32)

    @pl.loop(0, RPW // EB)
    def _(j):
        pltpu.sync_copy(zb, acc_sh.at[pl.ds(s * RPW + j * EB, EB)])

    plsc.subcore_barrier()

    for b in range(4):
        body(b, b % 4, b % 2, first=(b == 0))

    @pl.loop(4, NB - 4, step=4)
    def _(g):
        for jj in range(4):
            body(g + jj, jj, jj % 2)

    for b in range(NB - 4, NB):
        body(b, b % 4, b % 2,
             has_g=(b + 1 < NB), has_i=(b + 2 < NB), last=(b == NB - 1))

    plsc.subcore_barrier()
    row = pl.multiple_of(s * CPW, 8)
    pltpu.sync_copy(acc_sh.at[pl.ds(row, CPW)],
                    out_hbm.at[c, pl.ds(row, CPW)])


def _tc_front_a_body(x_ref, emb_ref, wl_ref, bl_ref, w1_ref, hw_ref):
    xb = x_ref[...]
    xt = xb[:, :NTYPES]
    m = jnp.max(xt, axis=1, keepdims=True)
    iota = lax.broadcasted_iota(jnp.int32, xt.shape, 1)
    idx = jnp.min(jnp.where(xt == m, iota, NTYPES), axis=1, keepdims=True)
    onehot = (iota == idx).astype(jnp.float32)
    table = jnp.dot(emb_ref[...], wl_ref[:EMB, :],
                    preferred_element_type=jnp.float32)
    h = jnp.dot(onehot, table, preferred_element_type=jnp.float32)
    h = h + jnp.dot(xb[:, NTYPES:], wl_ref[EMB:, :],
                    preferred_element_type=jnp.float32)
    h = jnp.maximum(h + bl_ref[...], 0.0)
    hw_ref[...] = jnp.dot(h, w1_ref[...], preferred_element_type=jnp.float32)


def _tc_front_a(x, atom_emb, W_lin, b_lin, W1):
    return pl.pallas_call(
        _tc_front_a_body,
        grid=(NRB,),
        in_specs=[
            pl.BlockSpec((RB, NTYPES + FIXED), lambda i: (i, 0)),
            pl.BlockSpec((NTYPES, EMB), lambda i: (0, 0)),
            pl.BlockSpec((EMB + FIXED, DIM), lambda i: (0, 0)),
            pl.BlockSpec((1, DIM), lambda i: (0, 0)),
            pl.BlockSpec((DIM, DIM), lambda i: (0, 0)),
        ],
        out_specs=pl.BlockSpec((RB, DIM), lambda i: (i, 0)),
        out_shape=jax.ShapeDtypeStruct((N, DIM), jnp.float32),
    )(x, atom_emb, W_lin, b_lin.reshape(1, DIM), W1)


def _tc_front_b_body(hist_ref, hw_ref, hws_ref, dinv_ref):
    deg = hist_ref[0, :, :1] + hist_ref[1, :, :1] + 1.0
    dinv = lax.rsqrt(deg)
    hws_ref[...] = hw_ref[...] * dinv
    dinv_ref[...] = dinv


def _tc_front_b(hist, hw):
    return pl.pallas_call(
        _tc_front_b_body,
        grid=(NRB,),
        in_specs=[
            pl.BlockSpec((NC, RB, L), lambda i: (0, i, 0)),
            pl.BlockSpec((RB, DIM), lambda i: (i, 0)),
        ],
        out_specs=[
            pl.BlockSpec((RB, DIM), lambda i: (i, 0)),
            pl.BlockSpec((RB, 1), lambda i: (i, 0)),
        ],
        out_shape=[
            jax.ShapeDtypeStruct((N, DIM), jnp.float32),
            jax.ShapeDtypeStruct((N, 1), jnp.float32),
        ],
    )(hist, hw)


def _tc_mid_body(a_ref, hws_ref, dinv_ref, b_ref, w_ref, out_ref):
    dinv = dinv_ref[...]
    h = dinv * (a_ref[0] + a_ref[1] + hws_ref[...]) + b_ref[...]
    h = jnp.maximum(h, 0.0)
    out_ref[...] = jnp.dot(h, w_ref[...], preferred_element_type=jnp.float32) * dinv


def _tc_mid(acc, hws, dinv, b, W):
    return pl.pallas_call(
        _tc_mid_body,
        grid=(NRB,),
        in_specs=[
            pl.BlockSpec((NC, RB, DIM), lambda i: (0, i, 0)),
            pl.BlockSpec((RB, DIM), lambda i: (i, 0)),
            pl.BlockSpec((RB, 1), lambda i: (i, 0)),
            pl.BlockSpec((1, DIM), lambda i: (0, 0)),
            pl.BlockSpec((DIM, DIM), lambda i: (0, 0)),
        ],
        out_specs=pl.BlockSpec((RB, DIM), lambda i: (i, 0)),
        out_shape=jax.ShapeDtypeStruct((N, DIM), jnp.float32),
    )(acc, hws, dinv, b.reshape(1, DIM), W)


def _tc_final_body(a_ref, hws_ref, dinv_ref, b_ref, out_ref):
    h = dinv_ref[...] * (a_ref[0] + a_ref[1] + hws_ref[...]) + b_ref[...]
    out_ref[...] = jnp.maximum(h, 0.0)


def _tc_final(acc, hws, dinv, b):
    return pl.pallas_call(
        _tc_final_body,
        grid=(NRB,),
        in_specs=[
            pl.BlockSpec((NC, RB, DIM), lambda i: (0, i, 0)),
            pl.BlockSpec((RB, DIM), lambda i: (i, 0)),
            pl.BlockSpec((RB, 1), lambda i: (i, 0)),
            pl.BlockSpec((1, DIM), lambda i: (0, 0)),
        ],
        out_specs=pl.BlockSpec((RB, DIM), lambda i: (i, 0)),
        out_shape=jax.ShapeDtypeStruct((N, DIM), jnp.float32),
    )(acc, hws, dinv, b.reshape(1, DIM))


def kernel(x, edge_index, batch, atom_emb, W_lin, b_lin, W1, b1, W2, b2):
    del batch
    pad = EPAD - E
    pad_src = (jnp.arange(pad, dtype=jnp.int32) * 127) % N
    pad_dst = N + (jnp.arange(pad, dtype=jnp.int32) % (NPAD - N))
    src_r = jnp.concatenate([edge_index[0], pad_src])
    dst_r = jnp.concatenate([edge_index[1], pad_dst])
    edges = jnp.stack([src_r.reshape(NW, NB, EB), dst_r.reshape(NW, NB, EB)],
                      axis=2)

    hw1 = _tc_front_a(x, atom_emb, W_lin, b_lin, W1)
    hist = _sc_degree(edges)
    hws1, dinv = _tc_front_b(hist, hw1)
    acc1 = _sc_aggregate(hws1, edges)
    hws2 = _tc_mid(acc1, hws1, dinv, b1, W2)
    acc2 = _sc_aggregate(hws2, edges)
    return _tc_final(acc2, hws2, dinv, b2)
